# baseline (speedup 1.0000x reference)
import functools

import jax
import jax.numpy as jnp
from jax import lax
from jax.experimental import pallas as pl
from jax.experimental.pallas import tpu as pltpu
from jax.experimental.pallas import tpu_sc as plsc


def _pad_to(v, m):
    return (v + m - 1) // m * m


def _t1_body(x_ref, w1_ref, b1_ref, d1_ref, out_ref, rd_ref):
    h = lax.dot_general(x_ref[:], w1_ref[:], (((1,), (1,)), ((), ())),
                        preferred_element_type=jnp.float32)
    r = jnp.maximum(h + b1_ref[:], 0.0)
    out_ref[:] = r
    rd = r * d1_ref[:]
    rd_ref[:] = jnp.concatenate([rd, jnp.zeros_like(rd)], axis=1)


def _t1(x_pad, W1, b1, d1c, bm):
    np_, d_feat = x_pad.shape
    hid = W1.shape[0]
    return pl.pallas_call(
        _t1_body,
        grid=(np_ // bm,),
        in_specs=[
            pl.BlockSpec((bm, d_feat), lambda i: (i, 0)),
            pl.BlockSpec((hid, d_feat), lambda i: (0, 0)),
            pl.BlockSpec((1, hid), lambda i: (0, 0)),
            pl.BlockSpec((bm, 1), lambda i: (i, 0)),
        ],
        out_specs=[pl.BlockSpec((bm, hid), lambda i: (i, 0)),
                   pl.BlockSpec((bm, 2 * hid), lambda i: (i, 0))],
        out_shape=[jax.ShapeDtypeStruct((np_, hid), jnp.float32),
                   jax.ShapeDtypeStruct((np_, 2 * hid), jnp.float32)],
    )(x_pad, W1, b1.reshape(1, hid), d1c)


def _t2_body(aik_ref, akj_ref, aij_ref, m2_ref, cs1_ref, cs2_ref,
             acc_ref, *, nk, bm, bn):
    j = pl.program_id(0)
    i = pl.program_id(1)
    k = pl.program_id(2)

    @pl.when(k == 0)
    def _():
        acc_ref[:] = jnp.zeros_like(acc_ref)

    acc_ref[:] += lax.dot_general(
        aik_ref[:], akj_ref[:], (((1,), (0,)), ((), ())),
        preferred_element_type=jnp.int32)

    @pl.when(k == nk - 1)
    def _():
        adj = aij_ref[:] != 0
        rows = i * bm + lax.broadcasted_iota(jnp.int32, (bm, bn), 0)
        cols = j * bn + lax.broadcasted_iota(jnp.int32, (bm, bn), 1)
        notdiag = rows != cols
        m1 = adj & notdiag
        m2 = (acc_ref[:] > 0) & (~adj) & notdiag
        m2_ref[:] = m2.astype(jnp.int8)
        c1 = jnp.sum(m1.astype(jnp.float32), axis=0, keepdims=True)
        c2 = jnp.sum(m2.astype(jnp.float32), axis=0, keepdims=True)

        @pl.when(i == 0)
        def _():
            cs1_ref[:] = c1
            cs2_ref[:] = c2

        @pl.when(i > 0)
        def _():
            cs1_ref[:] += c1
            cs2_ref[:] += c2


def _t2(A, bm, bn, bk):
    np_ = A.shape[0]
    nj, ni, nk = np_ // bn, np_ // bm, np_ // bk
    body = functools.partial(_t2_body, nk=nk, bm=bm, bn=bn)
    return pl.pallas_call(
        body,
        grid=(nj, ni, nk),
        in_specs=[
            pl.BlockSpec((bm, bk), lambda j, i, k: (i, k)),
            pl.BlockSpec((bk, bn), lambda j, i, k: (k, j)),
            pl.BlockSpec((bm, bn), lambda j, i, k: (i, j)),
        ],
        out_specs=[
            pl.BlockSpec((bm, bn), lambda j, i, k: (i, j)),
            pl.BlockSpec((1, bn), lambda j, i, k: (0, j)),
            pl.BlockSpec((1, bn), lambda j, i, k: (0, j)),
        ],
        out_shape=[
            jax.ShapeDtypeStruct((np_, np_), jnp.int8),
            jax.ShapeDtypeStruct((1, np_), jnp.float32),
            jax.ShapeDtypeStruct((1, np_), jnp.float32),
        ],
        scratch_shapes=[pltpu.VMEM((bm, bn), jnp.int32)],
        compiler_params=pltpu.CompilerParams(
            dimension_semantics=("arbitrary", "arbitrary", "arbitrary")),
    )(A, A, A)


_SC_CHUNK = 128


def _sc_spmm(rd, c1p, r1p, zeros, np_):
    w = rd.shape[1]
    info = plsc.get_sparse_core_info()
    nc, ns = info.num_cores, info.num_subcores
    nw = nc * ns
    e_pad = c1p.shape[0]
    ew = e_pad // nw
    nchunks = ew // _SC_CHUNK
    rows_per_sub = np_ // ns

    mesh = plsc.VectorSubcoreMesh(core_axis_name="c", subcore_axis_name="s")

    @functools.partial(
        pl.kernel, mesh=mesh,
        out_type=jax.ShapeDtypeStruct((nc, np_, w), jnp.float32),
        scratch_types=[
            pltpu.VMEM((_SC_CHUNK,), jnp.int32),
            pltpu.VMEM((_SC_CHUNK,), jnp.int32),
            pltpu.VMEM((_SC_CHUNK, w), jnp.float32),
            pltpu.VMEM_SHARED((np_, w), jnp.float32),
            pltpu.SemaphoreType.DMA,
        ],
    )
    def k(rd_hbm, c1_hbm, r1_hbm, z_hbm, out_hbm, cbuf, rbuf, rows, acc, sem):
        c = lax.axis_index("c")
        s = lax.axis_index("s")
        wid = s * nc + c
        sub_lo = s * rows_per_sub
        pltpu.sync_copy(z_hbm.at[pl.ds(sub_lo, rows_per_sub)],
                        acc.at[pl.ds(sub_lo, rows_per_sub)])
        plsc.subcore_barrier()
        base = wid * ew

        def body(i, carry):
            off = base + i * _SC_CHUNK
            pltpu.sync_copy(c1_hbm.at[pl.ds(off, _SC_CHUNK)], cbuf)
            pltpu.sync_copy(r1_hbm.at[pl.ds(off, _SC_CHUNK)], rbuf)
            pltpu.async_copy(rd_hbm.at[cbuf], rows, sem).wait()
            pltpu.sync_copy(rows, acc.at[rbuf], add=True)
            return carry

        lax.fori_loop(0, nchunks, body, 0)
        plsc.subcore_barrier()
        pltpu.sync_copy(acc.at[pl.ds(sub_lo, rows_per_sub)],
                        out_hbm.at[c, pl.ds(sub_lo, rows_per_sub)])

    return k(rd, c1p, r1p, zeros)


def _t3_body(m2_ref, r_ref, d2k_ref, d1i_ref, d2i_ref, p0_ref, p1_ref,
             out_ref, rd_ref, acc2_ref, *, nk, hid):
    k = pl.program_id(1)

    @pl.when(k == 0)
    def _():
        acc2_ref[:] = jnp.zeros_like(acc2_ref)

    rd2 = r_ref[:] * d2k_ref[:]
    acc2_ref[:] += lax.dot_general(
        m2_ref[:].astype(jnp.float32), rd2, (((1,), (0,)), ((), ())),
        preferred_element_type=jnp.float32)

    @pl.when(k == nk - 1)
    def _():
        o1 = d1i_ref[:] * (p0_ref[0][:, :hid] + p1_ref[0][:, :hid])
        o = jnp.maximum(o1, d2i_ref[:] * acc2_ref[:])
        out_ref[:] = o
        rd = o * d1i_ref[:]
        rd_ref[:] = jnp.concatenate([rd, jnp.zeros_like(rd)], axis=1)


def _t3(m2, R, P, d1c, d2c, bm, bk):
    np_, hid = R.shape
    pw = P.shape[2]
    ni, nk = np_ // bm, np_ // bk
    body = functools.partial(_t3_body, nk=nk, hid=hid)
    return pl.pallas_call(
        body,
        grid=(ni, nk),
        in_specs=[
            pl.BlockSpec((bm, bk), lambda i, k: (i, k)),
            pl.BlockSpec((bk, hid), lambda i, k: (k, 0)),
            pl.BlockSpec((bk, 1), lambda i, k: (k, 0)),
            pl.BlockSpec((bm, 1), lambda i, k: (i, 0)),
            pl.BlockSpec((bm, 1), lambda i, k: (i, 0)),
            pl.BlockSpec((1, bm, pw), lambda i, k: (0, i, 0)),
            pl.BlockSpec((1, bm, pw), lambda i, k: (1, i, 0)),
        ],
        out_specs=[pl.BlockSpec((bm, hid), lambda i, k: (i, 0)),
                   pl.BlockSpec((bm, 2 * hid), lambda i, k: (i, 0))],
        out_shape=[jax.ShapeDtypeStruct((np_, hid), jnp.float32),
                   jax.ShapeDtypeStruct((np_, 2 * hid), jnp.float32)],
        scratch_shapes=[pltpu.VMEM((bm, hid), jnp.float32)],
        compiler_params=pltpu.CompilerParams(
            dimension_semantics=("arbitrary", "arbitrary")),
    )(m2, R, d2c, d1c, d2c, P, P)


def _t4_body(r0_ref, r1_ref, r2_ref, w0_ref, w1_ref, w2_ref, b_ref, out_ref):
    logits = (
        lax.dot_general(r0_ref[:], w0_ref[:], (((1,), (0,)), ((), ())),
                        preferred_element_type=jnp.float32)
        + lax.dot_general(r1_ref[:], w1_ref[:], (((1,), (0,)), ((), ())),
                          preferred_element_type=jnp.float32)
        + lax.dot_general(r2_ref[:], w2_ref[:], (((1,), (0,)), ((), ())),
                          preferred_element_type=jnp.float32)
        + b_ref[:])
    m = jnp.max(logits, axis=1, keepdims=True)
    s = logits - m
    lse = jnp.log(jnp.sum(jnp.exp(s), axis=1, keepdims=True))
    out_ref[:] = s - lse


def _t4(R0, R1, R2, W2, b2, bm):
    np_, hid = R0.shape
    ncls = W2.shape[0]
    w0 = W2[:, 0 * hid:1 * hid].T
    w1 = W2[:, 1 * hid:2 * hid].T
    w2 = W2[:, 2 * hid:3 * hid].T
    return pl.pallas_call(
        _t4_body,
        grid=(np_ // bm,),
        in_specs=[
            pl.BlockSpec((bm, hid), lambda i: (i, 0)),
            pl.BlockSpec((bm, hid), lambda i: (i, 0)),
            pl.BlockSpec((bm, hid), lambda i: (i, 0)),
            pl.BlockSpec((hid, ncls), lambda i: (0, 0)),
            pl.BlockSpec((hid, ncls), lambda i: (0, 0)),
            pl.BlockSpec((hid, ncls), lambda i: (0, 0)),
            pl.BlockSpec((1, ncls), lambda i: (0, 0)),
        ],
        out_specs=pl.BlockSpec((bm, ncls), lambda i: (i, 0)),
        out_shape=jax.ShapeDtypeStruct((np_, ncls), jnp.float32),
    )(R0, R1, R2, w0, w1, w2, b2.reshape(1, ncls))


def kernel(x, edge_index, W1, b1, W2, b2):
    n, d_feat = x.shape
    hid = W1.shape[0]
    e = edge_index.shape[1]
    np_ = _pad_to(n, 2048)
    bm2 = bn2 = min(2048, np_)
    bk2 = min(512, np_)
    bm3 = min(512, np_)
    bk3 = min(2048, np_)
    bm14 = min(1024, np_)

    src = edge_index[0].astype(jnp.int32)
    dst = edge_index[1].astype(jnp.int32)

    codes = jnp.sort(src * n + dst)
    dup = jnp.concatenate(
        [jnp.zeros((1,), jnp.bool_), codes[1:] == codes[:-1]])
    invalid = dup | ((codes // n) == (codes % n))
    sent = n * n
    codes1 = jnp.sort(jnp.where(invalid, sent, codes))
    r1 = jnp.where(codes1 < sent, codes1 // n, n).astype(jnp.int32)
    c1 = jnp.where(codes1 < sent, codes1 % n, 0).astype(jnp.int32)
    e_pad = _pad_to(e, 32 * _SC_CHUNK)
    r1p = jnp.concatenate([r1, jnp.full((e_pad - e,), n, jnp.int32)])
    c1p = jnp.concatenate([c1, jnp.zeros((e_pad - e,), jnp.int32)])

    A = jnp.zeros((np_, np_), jnp.int8).at[src, dst].set(jnp.int8(1))

    m2, cs1, cs2 = _t2(A, bm2, bn2, bk2)
    d1 = jnp.where(cs1 > 0, lax.rsqrt(jnp.maximum(cs1, 1e-30)), 0.0)
    d2 = jnp.where(cs2 > 0, lax.rsqrt(jnp.maximum(cs2, 1e-30)), 0.0)
    d1c = d1.reshape(np_, 1)
    d2c = d2.reshape(np_, 1)

    x_pad = jnp.pad(x, ((0, np_ - n), (0, 0)))
    zeros = jnp.zeros((np_, 2 * hid), jnp.float32)

    R0, Rd0 = _t1(x_pad, W1, b1, d1c, bm14)
    P1 = _sc_spmm(Rd0, c1p, r1p, zeros, np_)
    R1, Rd1 = _t3(m2, R0, P1, d1c, d2c, bm3, bk3)
    P2 = _sc_spmm(Rd1, c1p, r1p, zeros, np_)
    R2, _ = _t3(m2, R1, P2, d1c, d2c, bm3, bk3)

    out = _t4(R0, R1, R2, W2, b2, bm14)
    return out[:n]


# speedup vs baseline: 1.0197x; 1.0197x over previous
import functools

import jax
import jax.numpy as jnp
from jax import lax
from jax.experimental import pallas as pl
from jax.experimental.pallas import tpu as pltpu
from jax.experimental.pallas import tpu_sc as plsc


def _pad_to(v, m):
    return (v + m - 1) // m * m


def _t1_body(x_ref, w1_ref, b1_ref, d1_ref, out_ref, rd_ref):
    h = lax.dot_general(x_ref[:], w1_ref[:], (((1,), (1,)), ((), ())),
                        preferred_element_type=jnp.float32)
    r = jnp.maximum(h + b1_ref[:], 0.0)
    out_ref[:] = r
    rd = r * d1_ref[:]
    rd_ref[:] = jnp.concatenate([rd, jnp.zeros_like(rd)], axis=1)


def _t1(x_pad, W1, b1, d1c, bm):
    np_, d_feat = x_pad.shape
    hid = W1.shape[0]
    return pl.pallas_call(
        _t1_body,
        grid=(np_ // bm,),
        in_specs=[
            pl.BlockSpec((bm, d_feat), lambda i: (i, 0)),
            pl.BlockSpec((hid, d_feat), lambda i: (0, 0)),
            pl.BlockSpec((1, hid), lambda i: (0, 0)),
            pl.BlockSpec((bm, 1), lambda i: (i, 0)),
        ],
        out_specs=[pl.BlockSpec((bm, hid), lambda i: (i, 0)),
                   pl.BlockSpec((bm, 2 * hid), lambda i: (i, 0))],
        out_shape=[jax.ShapeDtypeStruct((np_, hid), jnp.float32),
                   jax.ShapeDtypeStruct((np_, 2 * hid), jnp.float32)],
    )(x_pad, W1, b1.reshape(1, hid), d1c)


def _t2_body(aik_ref, akj_ref, aij_ref, m2_ref, cs1_ref, cs2_ref,
             acc_ref, *, nk, bm, bn):
    j = pl.program_id(0)
    i = pl.program_id(1)
    k = pl.program_id(2)

    @pl.when(k == 0)
    def _():
        acc_ref[:] = jnp.zeros_like(acc_ref)

    acc_ref[:] += lax.dot_general(
        aik_ref[:], akj_ref[:], (((1,), (0,)), ((), ())),
        preferred_element_type=jnp.int32)

    @pl.when(k == nk - 1)
    def _():
        adj = aij_ref[:] != 0
        rows = i * bm + lax.broadcasted_iota(jnp.int32, (bm, bn), 0)
        cols = j * bn + lax.broadcasted_iota(jnp.int32, (bm, bn), 1)
        notdiag = rows != cols
        m1 = adj & notdiag
        m2 = (acc_ref[:] > 0) & (~adj) & notdiag
        m2_ref[:] = m2.astype(jnp.int8)
        c1 = jnp.sum(m1.astype(jnp.float32), axis=0, keepdims=True)
        c2 = jnp.sum(m2.astype(jnp.float32), axis=0, keepdims=True)

        @pl.when(i == 0)
        def _():
            cs1_ref[:] = c1
            cs2_ref[:] = c2

        @pl.when(i > 0)
        def _():
            cs1_ref[:] += c1
            cs2_ref[:] += c2


def _t2(A, bm, bn, bk):
    np_ = A.shape[0]
    nj, ni, nk = np_ // bn, np_ // bm, np_ // bk
    body = functools.partial(_t2_body, nk=nk, bm=bm, bn=bn)
    return pl.pallas_call(
        body,
        grid=(nj, ni, nk),
        in_specs=[
            pl.BlockSpec((bm, bk), lambda j, i, k: (i, k)),
            pl.BlockSpec((bk, bn), lambda j, i, k: (k, j)),
            pl.BlockSpec((bm, bn), lambda j, i, k: (i, j)),
        ],
        out_specs=[
            pl.BlockSpec((bm, bn), lambda j, i, k: (i, j)),
            pl.BlockSpec((1, bn), lambda j, i, k: (0, j)),
            pl.BlockSpec((1, bn), lambda j, i, k: (0, j)),
        ],
        out_shape=[
            jax.ShapeDtypeStruct((np_, np_), jnp.int8),
            jax.ShapeDtypeStruct((1, np_), jnp.float32),
            jax.ShapeDtypeStruct((1, np_), jnp.float32),
        ],
        scratch_shapes=[pltpu.VMEM((bm, bn), jnp.int32)],
        compiler_params=pltpu.CompilerParams(
            dimension_semantics=("arbitrary", "arbitrary", "arbitrary")),
    )(A, A, A)


_SC_CHUNK = 128


def _sc_spmm(rd, c1p, r1p, zeros, np_):
    w = rd.shape[1]
    info = plsc.get_sparse_core_info()
    nc, ns = info.num_cores, info.num_subcores
    nw = nc * ns
    e_pad = c1p.shape[0]
    ew = e_pad // nw
    nchunks = ew // _SC_CHUNK
    rows_per_sub = np_ // ns

    ci = c1p.reshape(nw, nchunks, _SC_CHUNK)
    ri = r1p.reshape(nw, nchunks, _SC_CHUNK)

    mesh = plsc.VectorSubcoreMesh(core_axis_name="c", subcore_axis_name="s")

    @functools.partial(
        pl.kernel, mesh=mesh,
        out_type=jax.ShapeDtypeStruct((nc, np_, w), jnp.float32),
        scratch_types=[
            pltpu.VMEM((nchunks, _SC_CHUNK), jnp.int32),
            pltpu.VMEM((nchunks, _SC_CHUNK), jnp.int32),
            pltpu.VMEM((_SC_CHUNK, w), jnp.float32),
            pltpu.VMEM((_SC_CHUNK, w), jnp.float32),
            pltpu.VMEM_SHARED((np_, w), jnp.float32),
            pltpu.SemaphoreType.DMA,
            pltpu.SemaphoreType.DMA,
        ],
    )
    def k(rd_hbm, ci_hbm, ri_hbm, z_hbm, out_hbm, cbuf, rbuf, rows0, rows1,
          acc, sem0, sem1):
        c = lax.axis_index("c")
        s = lax.axis_index("s")
        wid = s * nc + c
        sub_lo = s * rows_per_sub
        pltpu.sync_copy(ci_hbm.at[wid], cbuf)
        pltpu.sync_copy(ri_hbm.at[wid], rbuf)
        pltpu.sync_copy(z_hbm.at[pl.ds(sub_lo, rows_per_sub)],
                        acc.at[pl.ds(sub_lo, rows_per_sub)])
        plsc.subcore_barrier()

        rows = (rows0, rows1)
        sems = (sem0, sem1)
        pltpu.make_async_copy(rd_hbm.at[cbuf.at[0]], rows0, sem0).start()

        def body(i2, carry):
            for b in range(2):
                ch = i2 * 2 + b
                pltpu.make_async_copy(
                    rd_hbm.at[cbuf.at[ch]], rows[b], sems[b]).wait()

                @pl.when(ch + 1 < nchunks)
                def _():
                    pltpu.make_async_copy(
                        rd_hbm.at[cbuf.at[ch + 1]], rows[1 - b],
                        sems[1 - b]).start()

                pltpu.sync_copy(rows[b], acc.at[rbuf.at[ch]], add=True)
            return carry

        lax.fori_loop(0, nchunks // 2, body, 0)
        plsc.subcore_barrier()
        pltpu.sync_copy(acc.at[pl.ds(sub_lo, rows_per_sub)],
                        out_hbm.at[c, pl.ds(sub_lo, rows_per_sub)])

    return k(rd, ci, ri, zeros)


def _t3_body(m2_ref, r_ref, d2k_ref, d1i_ref, d2i_ref, p0_ref, p1_ref,
             out_ref, rd_ref, acc2_ref, *, nk, hid):
    k = pl.program_id(1)

    @pl.when(k == 0)
    def _():
        acc2_ref[:] = jnp.zeros_like(acc2_ref)

    rd2 = r_ref[:] * d2k_ref[:]
    acc2_ref[:] += lax.dot_general(
        m2_ref[:].astype(jnp.float32), rd2, (((1,), (0,)), ((), ())),
        preferred_element_type=jnp.float32)

    @pl.when(k == nk - 1)
    def _():
        o1 = d1i_ref[:] * (p0_ref[0][:, :hid] + p1_ref[0][:, :hid])
        o = jnp.maximum(o1, d2i_ref[:] * acc2_ref[:])
        out_ref[:] = o
        rd = o * d1i_ref[:]
        rd_ref[:] = jnp.concatenate([rd, jnp.zeros_like(rd)], axis=1)


def _t3(m2, R, P, d1c, d2c, bm, bk):
    np_, hid = R.shape
    pw = P.shape[2]
    ni, nk = np_ // bm, np_ // bk
    body = functools.partial(_t3_body, nk=nk, hid=hid)
    return pl.pallas_call(
        body,
        grid=(ni, nk),
        in_specs=[
            pl.BlockSpec((bm, bk), lambda i, k: (i, k)),
            pl.BlockSpec((bk, hid), lambda i, k: (k, 0)),
            pl.BlockSpec((bk, 1), lambda i, k: (k, 0)),
            pl.BlockSpec((bm, 1), lambda i, k: (i, 0)),
            pl.BlockSpec((bm, 1), lambda i, k: (i, 0)),
            pl.BlockSpec((1, bm, pw), lambda i, k: (0, i, 0)),
            pl.BlockSpec((1, bm, pw), lambda i, k: (1, i, 0)),
        ],
        out_specs=[pl.BlockSpec((bm, hid), lambda i, k: (i, 0)),
                   pl.BlockSpec((bm, 2 * hid), lambda i, k: (i, 0))],
        out_shape=[jax.ShapeDtypeStruct((np_, hid), jnp.float32),
                   jax.ShapeDtypeStruct((np_, 2 * hid), jnp.float32)],
        scratch_shapes=[pltpu.VMEM((bm, hid), jnp.float32)],
        compiler_params=pltpu.CompilerParams(
            dimension_semantics=("arbitrary", "arbitrary")),
    )(m2, R, d2c, d1c, d2c, P, P)


def _t4_body(r0_ref, r1_ref, r2_ref, w0_ref, w1_ref, w2_ref, b_ref, out_ref):
    logits = (
        lax.dot_general(r0_ref[:], w0_ref[:], (((1,), (0,)), ((), ())),
                        preferred_element_type=jnp.float32)
        + lax.dot_general(r1_ref[:], w1_ref[:], (((1,), (0,)), ((), ())),
                          preferred_element_type=jnp.float32)
        + lax.dot_general(r2_ref[:], w2_ref[:], (((1,), (0,)), ((), ())),
                          preferred_element_type=jnp.float32)
        + b_ref[:])
    m = jnp.max(logits, axis=1, keepdims=True)
    s = logits - m
    lse = jnp.log(jnp.sum(jnp.exp(s), axis=1, keepdims=True))
    out_ref[:] = s - lse


def _t4(R0, R1, R2, W2, b2, bm):
    np_, hid = R0.shape
    ncls = W2.shape[0]
    w0 = W2[:, 0 * hid:1 * hid].T
    w1 = W2[:, 1 * hid:2 * hid].T
    w2 = W2[:, 2 * hid:3 * hid].T
    return pl.pallas_call(
        _t4_body,
        grid=(np_ // bm,),
        in_specs=[
            pl.BlockSpec((bm, hid), lambda i: (i, 0)),
            pl.BlockSpec((bm, hid), lambda i: (i, 0)),
            pl.BlockSpec((bm, hid), lambda i: (i, 0)),
            pl.BlockSpec((hid, ncls), lambda i: (0, 0)),
            pl.BlockSpec((hid, ncls), lambda i: (0, 0)),
            pl.BlockSpec((hid, ncls), lambda i: (0, 0)),
            pl.BlockSpec((1, ncls), lambda i: (0, 0)),
        ],
        out_specs=pl.BlockSpec((bm, ncls), lambda i: (i, 0)),
        out_shape=jax.ShapeDtypeStruct((np_, ncls), jnp.float32),
    )(R0, R1, R2, w0, w1, w2, b2.reshape(1, ncls))


def kernel(x, edge_index, W1, b1, W2, b2):
    n, d_feat = x.shape
    hid = W1.shape[0]
    e = edge_index.shape[1]
    np_ = _pad_to(n, 2048)
    bm2 = bn2 = min(2048, np_)
    bk2 = min(512, np_)
    bm3 = min(512, np_)
    bk3 = min(2048, np_)
    bm14 = min(1024, np_)

    src = edge_index[0].astype(jnp.int32)
    dst = edge_index[1].astype(jnp.int32)

    codes = jnp.sort(src * n + dst)
    dup = jnp.concatenate(
        [jnp.zeros((1,), jnp.bool_), codes[1:] == codes[:-1]])
    invalid = dup | ((codes // n) == (codes % n))
    sent = n * n
    codes1 = jnp.sort(jnp.where(invalid, sent, codes))
    r1 = jnp.where(codes1 < sent, codes1 // n, n).astype(jnp.int32)
    c1 = jnp.where(codes1 < sent, codes1 % n, 0).astype(jnp.int32)
    e_pad = _pad_to(e, 32 * _SC_CHUNK)
    r1p = jnp.concatenate([r1, jnp.full((e_pad - e,), n, jnp.int32)])
    c1p = jnp.concatenate([c1, jnp.zeros((e_pad - e,), jnp.int32)])

    A = jnp.zeros((np_, np_), jnp.int8).at[src, dst].set(jnp.int8(1))

    m2, cs1, cs2 = _t2(A, bm2, bn2, bk2)
    d1 = jnp.where(cs1 > 0, lax.rsqrt(jnp.maximum(cs1, 1e-30)), 0.0)
    d2 = jnp.where(cs2 > 0, lax.rsqrt(jnp.maximum(cs2, 1e-30)), 0.0)
    d1c = d1.reshape(np_, 1)
    d2c = d2.reshape(np_, 1)

    x_pad = jnp.pad(x, ((0, np_ - n), (0, 0)))
    zeros = jnp.zeros((np_, 2 * hid), jnp.float32)

    R0, Rd0 = _t1(x_pad, W1, b1, d1c, bm14)
    P1 = _sc_spmm(Rd0, c1p, r1p, zeros, np_)
    R1, Rd1 = _t3(m2, R0, P1, d1c, d2c, bm3, bk3)
    P2 = _sc_spmm(Rd1, c1p, r1p, zeros, np_)
    R2, _ = _t3(m2, R1, P2, d1c, d2c, bm3, bk3)

    out = _t4(R0, R1, R2, W2, b2, bm14)
    return out[:n]
